# Initial kernel scaffold; baseline (speedup 1.0000x reference)
import jax
import jax.numpy as jnp
from jax.experimental import pallas as pl


def kernel(x, edge_index, W1, b1, W2, b2, W3, b3):
    raise NotImplementedError("write your pallas kernel here")


# speedup vs baseline: 12.9597x; 12.9597x over previous
import functools

import jax
import jax.numpy as jnp
from jax import lax
from jax.experimental import pallas as pl
from jax.experimental.pallas import tpu as pltpu
from jax.experimental.pallas import tpu_sc as plsc

N_NODES = 10000
N_EDGES = 160000
E_CHUNK = 128
NC = 2
NS = 16

_SC_MESH = plsc.VectorSubcoreMesh(
    core_axis_name="c", subcore_axis_name="s", num_cores=NC, num_subcores=NS
)


def _deg_body(dst_hbm, out0_hbm, out1_hbm, dst_v, ones_v, zer_v, deg_sh, sem):
    c = lax.axis_index("c")
    s = lax.axis_index("s")
    del sem

    @pl.loop(0, E_CHUNK // 16)
    def _(i):
        ones_v[pl.ds(i * 16, 16)] = jnp.ones((16,), jnp.float32)

    @pl.when(s == 0)
    def _():
        @pl.loop(0, N_NODES // 16)
        def _(i):
            zer_v[pl.ds(i * 16, 16)] = jnp.zeros((16,), jnp.float32)

        pltpu.sync_copy(zer_v, deg_sh)

    plsc.subcore_barrier()

    nchunk = (N_EDGES // NC) // E_CHUNK
    base_e = c * (N_EDGES // NC)

    @pl.loop(0, (nchunk + NS - 1) // NS)
    def _(j):
        cid = s + j * NS

        @pl.when(cid < nchunk)
        def _():
            off = base_e + cid * E_CHUNK
            pltpu.sync_copy(dst_hbm.at[pl.ds(off, E_CHUNK)], dst_v)
            pltpu.sync_copy(ones_v, deg_sh.at[dst_v], add=True)

    plsc.subcore_barrier()

    @pl.when(jnp.logical_and(s == 0, c == 0))
    def _():
        pltpu.sync_copy(deg_sh, out0_hbm)

    @pl.when(jnp.logical_and(s == 0, c == 1))
    def _():
        pltpu.sync_copy(deg_sh, out1_hbm)


_deg_call = pl.kernel(
    _deg_body,
    out_type=(
        jax.ShapeDtypeStruct((N_NODES,), jnp.float32),
        jax.ShapeDtypeStruct((N_NODES,), jnp.float32),
    ),
    mesh=_SC_MESH,
    scratch_types=[
        pltpu.VMEM((E_CHUNK,), jnp.int32),
        pltpu.VMEM((E_CHUNK,), jnp.float32),
        pltpu.VMEM((N_NODES,), jnp.float32),
        pltpu.VMEM_SHARED((N_NODES,), jnp.float32),
        pltpu.SemaphoreType.DMA,
    ],
)


def _make_agg(width):
    rows_per_tile = 624
    tail_base = 15 * rows_per_tile + rows_per_tile

    def body(glo, ghi, src_hbm, dst_hbm, alo, ahi, src_v, dst_v, rows_v,
             acc_sh, sem):
        c = lax.axis_index("c")
        s = lax.axis_index("s")

        @pl.loop(0, E_CHUNK)
        def _(r):
            for i in range(width // 16):
                rows_v[r, pl.ds(i * 16, 16)] = jnp.zeros((16,), jnp.float32)

        base = s * rows_per_tile

        @pl.loop(0, 4)
        def _(k):
            pltpu.sync_copy(rows_v, acc_sh.at[pl.ds(base + k * E_CHUNK,
                                                    E_CHUNK)])

        pltpu.sync_copy(rows_v.at[pl.ds(0, 112)],
                        acc_sh.at[pl.ds(base + 512, 112)])

        @pl.when(s == NS - 1)
        def _():
            pltpu.sync_copy(rows_v.at[pl.ds(0, 16)],
                            acc_sh.at[pl.ds(tail_base, 16)])

        plsc.subcore_barrier()

        nchunk = N_EDGES // E_CHUNK

        @pl.loop(0, (nchunk + NS - 1) // NS)
        def _(j):
            cid = s + j * NS

            @pl.when(cid < nchunk)
            def _():
                off = cid * E_CHUNK
                pltpu.sync_copy(src_hbm.at[pl.ds(off, E_CHUNK)], src_v)
                pltpu.sync_copy(dst_hbm.at[pl.ds(off, E_CHUNK)], dst_v)

                @pl.when(c == 0)
                def _():
                    pltpu.async_copy(glo.at[src_v], rows_v, sem).wait()

                @pl.when(c == 1)
                def _():
                    pltpu.async_copy(ghi.at[src_v], rows_v, sem).wait()

                pltpu.sync_copy(rows_v, acc_sh.at[dst_v], add=True)

        plsc.subcore_barrier()

        tile_rows = pl.ds(base, rows_per_tile)
        tail_rows = pl.ds(tail_base, 16)

        @pl.when(c == 0)
        def _():
            pltpu.sync_copy(acc_sh.at[tile_rows], alo.at[tile_rows])

            @pl.when(s == NS - 1)
            def _():
                pltpu.sync_copy(acc_sh.at[tail_rows], alo.at[tail_rows])

        @pl.when(c == 1)
        def _():
            pltpu.sync_copy(acc_sh.at[tile_rows], ahi.at[tile_rows])

            @pl.when(s == NS - 1)
            def _():
                pltpu.sync_copy(acc_sh.at[tail_rows], ahi.at[tail_rows])

    return pl.kernel(
        body,
        out_type=(
            jax.ShapeDtypeStruct((N_NODES, width), jnp.float32),
            jax.ShapeDtypeStruct((N_NODES, width), jnp.float32),
        ),
        mesh=_SC_MESH,
        scratch_types=[
            pltpu.VMEM((E_CHUNK,), jnp.int32),
            pltpu.VMEM((E_CHUNK,), jnp.int32),
            pltpu.VMEM((E_CHUNK, width), jnp.float32),
            pltpu.VMEM_SHARED((N_NODES, width), jnp.float32),
            pltpu.SemaphoreType.DMA,
        ],
    )


_agg128 = _make_agg(128)


def _agg_pad_body(gp, src_hbm, dst_hbm, p0, p1, src_v, dst_v, rows_v,
                  acc_sh, sem):
    width = 128
    rows_per_tile = 624
    tail_base = 9984
    c = lax.axis_index("c")
    s = lax.axis_index("s")

    @pl.loop(0, E_CHUNK)
    def _(r):
        for i in range(width // 16):
            rows_v[r, pl.ds(i * 16, 16)] = jnp.zeros((16,), jnp.float32)

    base = s * rows_per_tile

    @pl.loop(0, 4)
    def _(k):
        pltpu.sync_copy(rows_v, acc_sh.at[pl.ds(base + k * E_CHUNK, E_CHUNK)])

    pltpu.sync_copy(rows_v.at[pl.ds(0, 112)],
                    acc_sh.at[pl.ds(base + 512, 112)])

    @pl.when(s == NS - 1)
    def _():
        pltpu.sync_copy(rows_v.at[pl.ds(0, 16)],
                        acc_sh.at[pl.ds(tail_base, 16)])

    plsc.subcore_barrier()

    nchunk = (N_EDGES // NC) // E_CHUNK
    base_e = c * (N_EDGES // NC)

    @pl.loop(0, (nchunk + NS - 1) // NS)
    def _(j):
        cid = s + j * NS

        @pl.when(cid < nchunk)
        def _():
            off = base_e + cid * E_CHUNK
            pltpu.sync_copy(src_hbm.at[pl.ds(off, E_CHUNK)], src_v)
            pltpu.sync_copy(dst_hbm.at[pl.ds(off, E_CHUNK)], dst_v)
            pltpu.async_copy(gp.at[src_v], rows_v, sem).wait()
            pltpu.sync_copy(rows_v, acc_sh.at[dst_v], add=True)

    plsc.subcore_barrier()

    tile_rows = pl.ds(base, rows_per_tile)
    tail_rows = pl.ds(tail_base, 16)

    @pl.when(c == 0)
    def _():
        pltpu.sync_copy(acc_sh.at[tile_rows], p0.at[tile_rows])

        @pl.when(s == NS - 1)
        def _():
            pltpu.sync_copy(acc_sh.at[tail_rows], p0.at[tail_rows])

    @pl.when(c == 1)
    def _():
        pltpu.sync_copy(acc_sh.at[tile_rows], p1.at[tile_rows])

        @pl.when(s == NS - 1)
        def _():
            pltpu.sync_copy(acc_sh.at[tail_rows], p1.at[tail_rows])


_agg_pad = pl.kernel(
    _agg_pad_body,
    out_type=(
        jax.ShapeDtypeStruct((N_NODES, 128), jnp.float32),
        jax.ShapeDtypeStruct((N_NODES, 128), jnp.float32),
    ),
    mesh=_SC_MESH,
    scratch_types=[
        pltpu.VMEM((E_CHUNK,), jnp.int32),
        pltpu.VMEM((E_CHUNK,), jnp.int32),
        pltpu.VMEM((E_CHUNK, 128), jnp.float32),
        pltpu.VMEM_SHARED((N_NODES, 128), jnp.float32),
        pltpu.SemaphoreType.DMA,
    ],
)


_BN = 2000


def _dis_of(dp_ref):
    deg = dp_ref[:, 0:1] + dp_ref[:, 1:2] + 1.0
    return lax.rsqrt(deg)


def _k1_body(dp_ref, x_ref, w_ref, glo_ref, ghi_ref):
    dis = _dis_of(dp_ref)
    h = jnp.dot(x_ref[...], w_ref[...], preferred_element_type=jnp.float32)
    g = h * dis
    half = g.shape[1] // 2
    glo_ref[...] = g[:, :half]
    ghi_ref[...] = g[:, half:]


def _mid_body(dp_ref, alo_ref, ahi_ref, glo_ref, ghi_ref, w_ref, b_ref,
              olo_ref, ohi_ref):
    dis = _dis_of(dp_ref)
    accf = jnp.concatenate([alo_ref[...], ahi_ref[...]], axis=1)
    gf = jnp.concatenate([glo_ref[...], ghi_ref[...]], axis=1)
    z = jnp.maximum(dis * (accf + gf) + b_ref[...], 0.0)
    h = jnp.dot(z, w_ref[...], preferred_element_type=jnp.float32)
    g = h * dis
    half = g.shape[1] // 2
    olo_ref[...] = g[:, :half]
    ohi_ref[...] = g[:, half:]


def _k3_body(dp_ref, alo_ref, ahi_ref, glo_ref, ghi_ref, w_ref, b_ref,
             out_ref):
    dis = _dis_of(dp_ref)
    accf = jnp.concatenate([alo_ref[...], ahi_ref[...]], axis=1)
    gf = jnp.concatenate([glo_ref[...], ghi_ref[...]], axis=1)
    z = jnp.maximum(dis * (accf + gf) + b_ref[...], 0.0)
    h = jnp.dot(z, w_ref[...], preferred_element_type=jnp.float32)
    g = h * dis
    out_ref[...] = jnp.concatenate([g, jnp.zeros_like(g)], axis=1)


def _k4_body(dp_ref, p0_ref, p1_ref, gp_ref, b_ref, out_ref):
    dis = _dis_of(dp_ref)
    dout = b_ref.shape[1]
    accf = p0_ref[...][:, :dout] + p1_ref[...][:, :dout]
    gf = gp_ref[...][:, :dout]
    z = dis * (accf + gf) + b_ref[...]
    m = jnp.max(z, axis=1, keepdims=True)
    out_ref[...] = z - m - jnp.log(jnp.sum(jnp.exp(z - m), axis=1,
                                           keepdims=True))


def _node_spec(width):
    return pl.BlockSpec((_BN, width), lambda i: (i, 0))


def _full_spec(shape):
    return pl.BlockSpec(shape, lambda i: (0, 0))


def _k1_call(dp, x, w):
    din, dout = w.shape
    half = dout // 2
    return pl.pallas_call(
        _k1_body,
        grid=(N_NODES // _BN,),
        in_specs=[_node_spec(2), _node_spec(din), _full_spec(w.shape)],
        out_specs=(_node_spec(half), _node_spec(half)),
        out_shape=(
            jax.ShapeDtypeStruct((N_NODES, half), jnp.float32),
            jax.ShapeDtypeStruct((N_NODES, half), jnp.float32),
        ),
    )(dp, x, w)


def _mid_call(dp, alo, ahi, glo, ghi, w, b):
    din, dout = w.shape
    hin = din // 2
    half = dout // 2
    return pl.pallas_call(
        _mid_body,
        grid=(N_NODES // _BN,),
        in_specs=[
            _node_spec(2),
            _node_spec(hin), _node_spec(hin),
            _node_spec(hin), _node_spec(hin),
            _full_spec(w.shape), _full_spec((1, din)),
        ],
        out_specs=(_node_spec(half), _node_spec(half)),
        out_shape=(
            jax.ShapeDtypeStruct((N_NODES, half), jnp.float32),
            jax.ShapeDtypeStruct((N_NODES, half), jnp.float32),
        ),
    )(dp, alo, ahi, glo, ghi, w, b.reshape(1, din))


def _k3_call(dp, alo, ahi, glo, ghi, w, b):
    din = w.shape[0]
    hin = din // 2
    return pl.pallas_call(
        _k3_body,
        grid=(N_NODES // _BN,),
        in_specs=[
            _node_spec(2),
            _node_spec(hin), _node_spec(hin),
            _node_spec(hin), _node_spec(hin),
            _full_spec(w.shape), _full_spec((1, din)),
        ],
        out_specs=_node_spec(128),
        out_shape=jax.ShapeDtypeStruct((N_NODES, 128), jnp.float32),
    )(dp, alo, ahi, glo, ghi, w, b.reshape(1, din))


def _k4_call(dp, p0, p1, gp, b):
    dout = b.shape[0]
    return pl.pallas_call(
        _k4_body,
        grid=(N_NODES // _BN,),
        in_specs=[
            _node_spec(2),
            _node_spec(128), _node_spec(128), _node_spec(128),
            _full_spec((1, dout)),
        ],
        out_specs=_node_spec(dout),
        out_shape=jax.ShapeDtypeStruct((N_NODES, dout), jnp.float32),
    )(dp, p0, p1, gp, b.reshape(1, dout))


def kernel(x, edge_index, W1, b1, W2, b2, W3, b3):
    src = edge_index[0]
    dst = edge_index[1]

    deg0, deg1 = _deg_call(dst)
    dp = jnp.stack([deg0, deg1], axis=1)

    g1lo, g1hi = _k1_call(dp, x, W1)
    a1lo, a1hi = _agg128(g1lo, g1hi, src, dst)
    g2lo, g2hi = _mid_call(dp, a1lo, a1hi, g1lo, g1hi, W2, b1)
    a2lo, a2hi = _agg128(g2lo, g2hi, src, dst)
    g3p = _k3_call(dp, a2lo, a2hi, g2lo, g2hi, W3, b2)
    p0, p1 = _agg_pad(g3p, src, dst)
    return _k4_call(dp, p0, p1, g3p, b3)
